# baseline (speedup 1.0000x reference)
import jax
import jax.numpy as jnp
from jax.experimental import pallas as pl

_RATE = 0.1
_KEEP = 1.0 - _RATE
_SEED = 42

_K0 = 0
_K1 = _SEED
_K2 = _K0 ^ _K1 ^ 0x1BD11BDA

_ROT_A = (13, 15, 26, 6)
_ROT_B = (17, 29, 16, 24)
_KS = (_K0, _K1, _K2)

_CHUNK = 524288


def _threefry_bits(lo):
    x0 = jnp.full_like(lo, jnp.uint32(_K0))
    x1 = lo + jnp.uint32(_K1)
    for i in range(5):
        rots = _ROT_A if i % 2 == 0 else _ROT_B
        for r in rots:
            x0 = x0 + x1
            x1 = (x1 << jnp.uint32(r)) | (x1 >> jnp.uint32(32 - r))
            x1 = x1 ^ x0
        x0 = x0 + jnp.uint32(_KS[(i + 1) % 3])
        x1 = x1 + jnp.uint32(_KS[(i + 2) % 3] + i + 1)
    return x0 ^ x1


def _body(v_ref, o_ref):
    pid = pl.program_id(0)
    rows, cols = _CHUNK // 1024, 1024
    row = jax.lax.broadcasted_iota(jnp.uint32, (rows, cols), 0)
    col = jax.lax.broadcasted_iota(jnp.uint32, (rows, cols), 1)
    idx = (row * jnp.uint32(cols) + col
           + jnp.uint32(pid) * jnp.uint32(_CHUNK))
    bits = _threefry_bits(idx)
    fbits = (bits >> jnp.uint32(9)) | jnp.uint32(0x3F800000)
    u = jax.lax.bitcast_convert_type(fbits, jnp.float32) - jnp.float32(1.0)
    keep = u < jnp.float32(_KEEP)
    v2 = v_ref[...].reshape(rows, cols)
    out = jnp.where(keep, v2 / jnp.float32(_KEEP), jnp.float32(0.0))
    o_ref[...] = out.reshape(_CHUNK)


def kernel(values, indices):
    nnz = values.shape[0]
    out = pl.pallas_call(
        _body,
        grid=(pl.cdiv(nnz, _CHUNK),),
        in_specs=[pl.BlockSpec((_CHUNK,), lambda i: (i,))],
        out_specs=pl.BlockSpec((_CHUNK,), lambda i: (i,)),
        out_shape=jax.ShapeDtypeStruct((nnz,), jnp.float32),
    )(values)
    return out, indices


# speedup vs baseline: 1.1069x; 1.1069x over previous
import jax
import jax.numpy as jnp
from jax.experimental import pallas as pl

_RATE = 0.1
_KEEP = 1.0 - _RATE
_SEED = 42

_K0 = 0
_K1 = _SEED
_K2 = _K0 ^ _K1 ^ 0x1BD11BDA

_ROT_A = (13, 15, 26, 6)
_ROT_B = (17, 29, 16, 24)
_KS = (_K0, _K1, _K2)

_CHUNK = 131072


def _threefry_bits(lo):
    x0 = jnp.full_like(lo, jnp.uint32(_K0))
    x1 = lo + jnp.uint32(_K1)
    for i in range(5):
        rots = _ROT_A if i % 2 == 0 else _ROT_B
        for r in rots:
            x0 = x0 + x1
            x1 = (x1 << jnp.uint32(r)) | (x1 >> jnp.uint32(32 - r))
            x1 = x1 ^ x0
        x0 = x0 + jnp.uint32(_KS[(i + 1) % 3])
        x1 = x1 + jnp.uint32(_KS[(i + 2) % 3] + i + 1)
    return x0 ^ x1


def _body(v_ref, o_ref):
    pid = pl.program_id(0)
    rows, cols = _CHUNK // 1024, 1024
    row = jax.lax.broadcasted_iota(jnp.uint32, (rows, cols), 0)
    col = jax.lax.broadcasted_iota(jnp.uint32, (rows, cols), 1)
    idx = (row * jnp.uint32(cols) + col
           + jnp.uint32(pid) * jnp.uint32(_CHUNK))
    bits = _threefry_bits(idx)
    fbits = (bits >> jnp.uint32(9)) | jnp.uint32(0x3F800000)
    u = jax.lax.bitcast_convert_type(fbits, jnp.float32) - jnp.float32(1.0)
    keep = u < jnp.float32(_KEEP)
    v2 = v_ref[...].reshape(rows, cols)
    out = jnp.where(keep, v2 / jnp.float32(_KEEP), jnp.float32(0.0))
    o_ref[...] = out.reshape(_CHUNK)


def kernel(values, indices):
    nnz = values.shape[0]
    out = pl.pallas_call(
        _body,
        grid=(pl.cdiv(nnz, _CHUNK),),
        in_specs=[pl.BlockSpec((_CHUNK,), lambda i: (i,))],
        out_specs=pl.BlockSpec((_CHUNK,), lambda i: (i,)),
        out_shape=jax.ShapeDtypeStruct((nnz,), jnp.float32),
    )(values)
    return out, indices


# speedup vs baseline: 1.1125x; 1.0051x over previous
import jax
import jax.numpy as jnp
from jax.experimental import pallas as pl

_RATE = 0.1
_KEEP = 1.0 - _RATE
_SEED = 42

_K0 = 0
_K1 = _SEED
_K2 = _K0 ^ _K1 ^ 0x1BD11BDA

_ROT_A = (13, 15, 26, 6)
_ROT_B = (17, 29, 16, 24)
_KS = (_K0, _K1, _K2)

_CHUNK = 65536


def _threefry_bits(lo):
    x0 = jnp.full_like(lo, jnp.uint32(_K0))
    x1 = lo + jnp.uint32(_K1)
    for i in range(5):
        rots = _ROT_A if i % 2 == 0 else _ROT_B
        for r in rots:
            x0 = x0 + x1
            x1 = (x1 << jnp.uint32(r)) | (x1 >> jnp.uint32(32 - r))
            x1 = x1 ^ x0
        x0 = x0 + jnp.uint32(_KS[(i + 1) % 3])
        x1 = x1 + jnp.uint32(_KS[(i + 2) % 3] + i + 1)
    return x0 ^ x1


def _body(v_ref, o_ref):
    pid = pl.program_id(0)
    rows, cols = _CHUNK // 1024, 1024
    row = jax.lax.broadcasted_iota(jnp.uint32, (rows, cols), 0)
    col = jax.lax.broadcasted_iota(jnp.uint32, (rows, cols), 1)
    idx = (row * jnp.uint32(cols) + col
           + jnp.uint32(pid) * jnp.uint32(_CHUNK))
    bits = _threefry_bits(idx)
    fbits = (bits >> jnp.uint32(9)) | jnp.uint32(0x3F800000)
    u = jax.lax.bitcast_convert_type(fbits, jnp.float32) - jnp.float32(1.0)
    keep = u < jnp.float32(_KEEP)
    v2 = v_ref[...].reshape(rows, cols)
    out = jnp.where(keep, v2 / jnp.float32(_KEEP), jnp.float32(0.0))
    o_ref[...] = out.reshape(_CHUNK)


def kernel(values, indices):
    nnz = values.shape[0]
    out = pl.pallas_call(
        _body,
        grid=(pl.cdiv(nnz, _CHUNK),),
        in_specs=[pl.BlockSpec((_CHUNK,), lambda i: (i,))],
        out_specs=pl.BlockSpec((_CHUNK,), lambda i: (i,)),
        out_shape=jax.ShapeDtypeStruct((nnz,), jnp.float32),
    )(values)
    return out, indices
